# baseline (speedup 1.0000x reference)
import functools

import jax
import jax.numpy as jnp
from jax import lax
from jax.experimental import pallas as pl
from jax.experimental.pallas import tpu as pltpu
from jax.experimental.pallas import tpu_sc as plsc

B, N, F = 1024, 128, 9
H = 128
NF = F + 2
S = 4


def _build_sc_call():
    info = plsc.get_sparse_core_info()
    n_cores, n_sub = info.num_cores, info.num_subcores
    nw = n_cores * n_sub
    bpw = B // nw
    mesh = plsc.VectorSubcoreMesh(core_axis_name="c", subcore_axis_name="s")

    @functools.partial(
        pl.kernel,
        mesh=mesh,
        out_type=jax.ShapeDtypeStruct((B, N + 1, H), jnp.float32),
        compiler_params=pltpu.CompilerParams(use_tc_tiling_on_sc=False),
        scratch_types=(
            [pltpu.VMEM((S, NF, N), jnp.int32)]
            + [pltpu.VMEM((N + 1, H), jnp.float32) for _ in range(S)]
            + [pltpu.SemaphoreType.DMA for _ in range(3 * S + 1)]
        ),
    )
    def sc_call(idx_hbm, atom_hbm, indW_hbm, outdW_hbm, tok_hbm, out_hbm,
                idx_v, *rest):
        bufs = rest[:S]
        isems = rest[S:2 * S]
        asems = rest[2 * S:3 * S]
        osems = rest[3 * S:4 * S]
        gsem = rest[4 * S]
        wid = lax.axis_index("s") * n_cores + lax.axis_index("c")
        base = wid * bpw

        def fire_idx(s, b):
            pltpu.async_copy(idx_hbm.at[b], idx_v.at[s], isems[s])

        def wait_idx(s, b):
            pltpu.make_async_copy(idx_hbm.at[b], idx_v.at[s], isems[s]).wait()

        def fire_plane0(s):
            pltpu.async_copy(atom_hbm.at[idx_v.at[s, 0]],
                             bufs[s].at[pl.ds(1, N)], gsem)

        def wait_plane0(s):
            pltpu.make_async_copy(atom_hbm.at[idx_v.at[s, 0]],
                                  bufs[s].at[pl.ds(1, N)], gsem).wait()

        def fire_adds(s):
            dst = bufs[s].at[pl.ds(1, N)]
            for j in range(1, F):
                pltpu.async_copy(atom_hbm.at[idx_v.at[s, j]], dst, asems[s],
                                 add=True)
            pltpu.async_copy(indW_hbm.at[idx_v.at[s, F]], dst, asems[s],
                             add=True)
            pltpu.async_copy(outdW_hbm.at[idx_v.at[s, F + 1]], dst, asems[s],
                             add=True)

        def wait_adds(s):
            dst = bufs[s].at[pl.ds(1, N)]
            for j in range(1, F):
                pltpu.make_async_copy(atom_hbm.at[idx_v.at[s, j]], dst,
                                      asems[s]).wait()
            pltpu.make_async_copy(indW_hbm.at[idx_v.at[s, F]], dst,
                                  asems[s]).wait()
            pltpu.make_async_copy(outdW_hbm.at[idx_v.at[s, F + 1]], dst,
                                  asems[s]).wait()

        def fire_out(s, b):
            pltpu.async_copy(bufs[s], out_hbm.at[b], osems[s])

        def wait_out(s, b):
            pltpu.make_async_copy(bufs[s], out_hbm.at[b], osems[s]).wait()

        for s in range(S):
            pltpu.sync_copy(tok_hbm, bufs[s].at[pl.ds(0, 1)])

        def step(s, i, first):
            sp1 = (s + 1) % S
            sp2 = (s + 2) % S
            wait_plane0(s)
            fire_adds(s)
            wait_adds(sp2)
            fire_out(sp2, i - 2)
            fire_idx(sp2, jnp.minimum(i + 2, B - 1))
            if not first:
                wait_out(sp1, i - 3)
            wait_idx(sp1, jnp.minimum(i + 1, B - 1))
            fire_plane0(sp1)

        fire_idx(0, base)
        fire_idx(1, base + 1)
        fire_idx(2, base + 2)
        wait_idx(0, base)
        fire_plane0(0)
        wait_plane0(0)
        fire_adds(0)
        wait_idx(1, base + 1)
        fire_plane0(1)
        wait_plane0(1)
        fire_adds(1)
        wait_idx(2, base + 2)
        fire_plane0(2)
        fire_idx(3, base + 3)
        step(2, base + 2, first=True)

        def round_body(r, carry):
            t0 = base + 3 + r * S
            for u in range(S):
                step((3 + u) % S, t0 + u, first=False)
            return carry

        lax.fori_loop(0, (bpw - S) // S, round_body, 0)

        last = base + bpw - 1
        wait_plane0(3)
        fire_adds(3)
        wait_adds(1)
        fire_out(1, last - 2)
        wait_adds(2)
        fire_out(2, last - 1)
        wait_adds(3)
        fire_out(3, last)
        wait_out(0, last - 3)
        wait_out(1, last - 2)
        wait_out(2, last - 1)
        wait_out(3, last)
        wait_idx(0, B - 1)

    return sc_call


@jax.jit
def kernel(x, in_degree, out_degree, atom_W, in_degree_W, out_degree_W,
           graph_token_W):
    idx_all = jnp.concatenate(
        [x.transpose(0, 2, 1),
         in_degree[:, None, :],
         out_degree[:, None, :]], axis=1)
    sc_call = _build_sc_call()
    return sc_call(idx_all, atom_W, in_degree_W, out_degree_W, graph_token_W)


# speedup vs baseline: 1.2305x; 1.2305x over previous
import functools

import jax
import jax.numpy as jnp
from jax import lax
from jax.experimental import pallas as pl
from jax.experimental.pallas import tpu as pltpu
from jax.experimental.pallas import tpu_sc as plsc

B, N, F = 1024, 128, 9
H = 128
NF = F + 2


def _build_sc_call():
    info = plsc.get_sparse_core_info()
    n_cores, n_sub = info.num_cores, info.num_subcores
    nw = n_cores * n_sub
    bpw = B // nw
    mesh = plsc.VectorSubcoreMesh(core_axis_name="c", subcore_axis_name="s")

    @functools.partial(
        pl.kernel,
        mesh=mesh,
        out_type=jax.ShapeDtypeStruct((B, N + 1, H), jnp.float32),
        scratch_types=[
            pltpu.VMEM((NF, N), jnp.int32),
            pltpu.VMEM((NF, N), jnp.int32),
            pltpu.VMEM((N + 1, H), jnp.float32),
            pltpu.VMEM((N + 1, H), jnp.float32),
            pltpu.SemaphoreType.DMA,
            pltpu.SemaphoreType.DMA,
            pltpu.SemaphoreType.DMA,
            pltpu.SemaphoreType.DMA,
            pltpu.SemaphoreType.DMA,
            pltpu.SemaphoreType.DMA,
        ],
    )
    def sc_call(idx_hbm, atom_hbm, ind_hbm, outd_hbm, tok_hbm, out_hbm,
                idx0, idx1, buf0, buf1, isem0, isem1, g0, g1, a0, a1):
        wid = lax.axis_index("s") * n_cores + lax.axis_index("c")
        base = wid * bpw
        idxs = (idx0, idx1)
        bufs = (buf0, buf1)
        isems = (isem0, isem1)
        gsems = (g0, g1)
        asems = (a0, a1)

        def fire_adds(s):
            dst = bufs[s].at[pl.ds(1, N)]
            cps = [pltpu.async_copy(atom_hbm.at[idxs[s].at[j]], dst,
                                    asems[s], add=True)
                   for j in range(1, F)]
            cps.append(pltpu.async_copy(ind_hbm.at[idxs[s].at[F]], dst,
                                        asems[s], add=True))
            cps.append(pltpu.async_copy(outd_hbm.at[idxs[s].at[F + 1]], dst,
                                        asems[s], add=True))
            return cps

        def wait_plane0(s):
            pltpu.make_async_copy(atom_hbm.at[idxs[s].at[0]],
                                  bufs[s].at[pl.ds(1, N)], gsems[s]).wait()

        def fire_plane0(s):
            pltpu.async_copy(atom_hbm.at[idxs[s].at[0]],
                             bufs[s].at[pl.ds(1, N)], gsems[s])

        pltpu.sync_copy(tok_hbm, buf0.at[pl.ds(0, 1)])
        pltpu.sync_copy(tok_hbm, buf1.at[pl.ds(0, 1)])

        pltpu.sync_copy(idx_hbm.at[base], idx0)
        fire_plane0(0)

        def pair_body(k, carry):
            i0 = base + 2 * k
            i1 = i0 + 1
            i2 = jnp.minimum(i0 + 2, B - 1)
            cpi1 = pltpu.async_copy(idx_hbm.at[i1], idx1, isem1)
            wait_plane0(0)
            cps0 = fire_adds(0)
            cpi1.wait()
            fire_plane0(1)
            wait_plane0(1)
            cps1 = fire_adds(1)
            for cp in cps0:
                cp.wait()
            cpi0 = pltpu.async_copy(idx_hbm.at[i2], idx0, isem0)
            pltpu.sync_copy(buf0, out_hbm.at[i0])
            cpi0.wait()
            fire_plane0(0)
            for cp in cps1:
                cp.wait()
            pltpu.sync_copy(buf1, out_hbm.at[i1])
            return carry

        lax.fori_loop(0, bpw // 2, pair_body, 0)
        wait_plane0(0)

    return sc_call


@jax.jit
def kernel(x, in_degree, out_degree, atom_W, in_degree_W, out_degree_W,
           graph_token_W):
    idx_all = jnp.concatenate(
        [x.transpose(0, 2, 1),
         in_degree[:, None, :],
         out_degree[:, None, :]], axis=1)
    sc_call = _build_sc_call()
    return sc_call(idx_all, atom_W, in_degree_W, out_degree_W, graph_token_W)
